# Initial kernel scaffold; baseline (speedup 1.0000x reference)
import jax
import jax.numpy as jnp
from jax.experimental import pallas as pl


def kernel(x, edge_index, batch, layer_weights, lin_W, lin_b, lin_g, lin_be, conv_W, conv_b, conv_g, conv_be):
    raise NotImplementedError("write your pallas kernel here")


# speedup vs baseline: 4.5115x; 4.5115x over previous
import functools

import jax
import jax.numpy as jnp
from jax import lax
from jax.experimental import pallas as pl
from jax.experimental.pallas import tpu as pltpu
from jax.experimental.pallas import tpu_sc as plsc

N = 10000
E = 320000
D = 128
T = 10
G = 64

NC = 2
NS = 16
NW = NC * NS
EPW = E // NW
C = 80
NCHUNK = EPW // C
RPT = 624
TAIL = N - NS * RPT


def _seg_body(src_hbm, dst_hbm, zeros_hbm, h_hbm, out_hbm,
              src_v, dst_v, rows_v, acc_sh, sem):
    cid = lax.axis_index("c")
    sid = lax.axis_index("s")
    wid = sid * NC + cid

    pltpu.sync_copy(zeros_hbm.at[pl.ds(sid * RPT, RPT)],
                    acc_sh.at[pl.ds(sid * RPT, RPT)])

    @pl.when(sid == 0)
    def _():
        pltpu.sync_copy(zeros_hbm.at[pl.ds(NS * RPT, TAIL)],
                        acc_sh.at[pl.ds(NS * RPT, TAIL)])

    plsc.subcore_barrier()

    base = wid * EPW

    def body(j, _):
        off = base + j * C
        pltpu.sync_copy(src_hbm.at[pl.ds(off, C)], src_v)
        pltpu.sync_copy(dst_hbm.at[pl.ds(off, C)], dst_v)
        pltpu.async_copy(h_hbm.at[src_v], rows_v, sem).wait()
        pltpu.sync_copy(rows_v, acc_sh.at[dst_v], add=True)
        return 0

    lax.fori_loop(0, NCHUNK, body, 0)

    plsc.subcore_barrier()
    pltpu.sync_copy(acc_sh.at[pl.ds(sid * RPT, RPT)],
                    out_hbm.at[cid, pl.ds(sid * RPT, RPT)])

    @pl.when(sid == 0)
    def _():
        pltpu.sync_copy(acc_sh.at[pl.ds(NS * RPT, TAIL)],
                        out_hbm.at[cid, pl.ds(NS * RPT, TAIL)])


@functools.cache
def _seg_partials():
    return pl.kernel(
        _seg_body,
        out_type=jax.ShapeDtypeStruct((NC, N, D), jnp.float32),
        mesh=plsc.VectorSubcoreMesh(core_axis_name="c", subcore_axis_name="s",
                                    num_cores=NC, num_subcores=NS),
        scratch_types=[
            pltpu.VMEM((C,), jnp.int32),
            pltpu.VMEM((C,), jnp.int32),
            pltpu.VMEM((C, D), jnp.float32),
            pltpu.VMEM_SHARED((N, D), jnp.float32),
            pltpu.SemaphoreType.DMA,
        ],
    )


def _bn_elu(y, g, be):
    mean = jnp.mean(y, axis=0, keepdims=True)
    yc = y - mean
    var = jnp.mean(yc * yc, axis=0, keepdims=True)
    yn = yc * lax.rsqrt(var + 1e-5) * g + be
    return jnp.where(yn > 0, yn, jnp.exp(yn) - 1.0)


def _conv_body(x_ref, p_ref, W_ref, b_ref, g_ref, be_ref, o_ref):
    hh = x_ref[...] + p_ref[0] + p_ref[1]
    y = jnp.dot(hh, W_ref[...], preferred_element_type=jnp.float32) + b_ref[...]
    o_ref[...] = _bn_elu(y, g_ref[...], be_ref[...])


def _lin_z(h, W, b, g, be):
    y = jnp.dot(h, W, preferred_element_type=jnp.float32) + b
    return _bn_elu(y, g, be)


def _pool_max(z, mask):
    cols = []
    for t in range(T):
        v = jnp.where(mask, z[:, t:t + 1], -jnp.inf)
        cols.append(jnp.max(v, axis=0).reshape(G, 1))
    return jnp.concatenate(cols, axis=1)


def _heads_body(x_ref, x1_ref, x2_ref, batch_ref, lw_ref,
                lW_ref, lb_ref, lg_ref, lbe_ref,
                out_ref, Z_ref):
    z0 = _lin_z(x_ref[...], lW_ref[0], lb_ref[0], lg_ref[0], lbe_ref[0])
    z1 = _lin_z(x1_ref[...], lW_ref[1], lb_ref[1], lg_ref[1], lbe_ref[1])
    z2 = _lin_z(x2_ref[...], lW_ref[2], lb_ref[2], lg_ref[2], lbe_ref[2])
    z1 = z1 * lw_ref[1:2, 0:1]
    z2 = z2 * lw_ref[2:3, 0:1]

    Z_ref[...] = z0 * lw_ref[0:1, 0:1] + z1 + z2

    gids = lax.broadcasted_iota(jnp.int32, (1, G), 1)
    mask = batch_ref[...] == gids
    out_ref[...] = _pool_max(z0, mask) + _pool_max(z1, mask) + _pool_max(z2, mask)


def kernel(x, edge_index, batch, layer_weights, lin_W, lin_b, lin_g, lin_be,
           conv_W, conv_b, conv_g, conv_be):
    src = edge_index[0]
    dst = edge_index[1]
    zeros = jnp.zeros((N, D), jnp.float32)

    p0 = _seg_partials()(src, dst, zeros, x)

    x1 = pl.pallas_call(
        _conv_body,
        out_shape=jax.ShapeDtypeStruct((N, D), jnp.float32),
    )(x, p0, conv_W[0], conv_b[0].reshape(1, D), conv_g[0].reshape(1, D),
      conv_be[0].reshape(1, D))

    p1 = _seg_partials()(src, dst, zeros, x1)

    x2 = pl.pallas_call(
        _conv_body,
        out_shape=jax.ShapeDtypeStruct((N, D), jnp.float32),
    )(x1, p1, conv_W[1], conv_b[1].reshape(1, D), conv_g[1].reshape(1, D),
      conv_be[1].reshape(1, D))

    out, Z = pl.pallas_call(
        _heads_body,
        out_shape=[
            jax.ShapeDtypeStruct((G, T), jnp.float32),
            jax.ShapeDtypeStruct((N, T), jnp.float32),
        ],
    )(x, x1, x2, batch.reshape(N, 1), layer_weights.reshape(3, 1),
      lin_W, lin_b.reshape(3, 1, T), lin_g.reshape(3, 1, T),
      lin_be.reshape(3, 1, T))

    return out, Z, x2


# speedup vs baseline: 9.1431x; 2.0266x over previous
import functools

import jax
import jax.numpy as jnp
from jax import lax
from jax.experimental import pallas as pl
from jax.experimental.pallas import tpu as pltpu
from jax.experimental.pallas import tpu_sc as plsc

N = 10000
E = 320000
D = 128
T = 10
G = 64

NC = 2
NS = 16
NW = NC * NS
EPW = E // NW
C = 80
NCHUNK = EPW // C
RPT = 624
TAIL = N - NS * RPT


def _seg_body(src_hbm, dst_hbm, zeros_hbm, h_hbm, out_hbm,
              src_all, dst_b, rows_b, acc_sh, gsems, isems):
    cid = lax.axis_index("c")
    sid = lax.axis_index("s")
    wid = sid * NC + cid

    pltpu.sync_copy(zeros_hbm.at[pl.ds(sid * RPT, RPT)],
                    acc_sh.at[pl.ds(sid * RPT, RPT)])

    @pl.when(sid == 0)
    def _():
        pltpu.sync_copy(zeros_hbm.at[pl.ds(NS * RPT, TAIL)],
                        acc_sh.at[pl.ds(NS * RPT, TAIL)])

    base = wid * EPW
    pltpu.sync_copy(src_hbm.at[pl.ds(base, EPW)], src_all)
    plsc.subcore_barrier()

    def issue(c, b):
        pltpu.async_copy(dst_hbm.at[pl.ds(base + c * C, C)], dst_b.at[b],
                         isems.at[b])
        pltpu.async_copy(h_hbm.at[src_all.at[pl.ds(c * C, C)]], rows_b.at[b],
                         gsems.at[b])

    def finish(c, b):
        pltpu.make_async_copy(dst_hbm.at[pl.ds(base + c * C, C)], dst_b.at[b],
                              isems.at[b]).wait()
        pltpu.make_async_copy(h_hbm.at[src_all.at[pl.ds(c * C, C)]],
                              rows_b.at[b], gsems.at[b]).wait()
        pltpu.sync_copy(rows_b.at[b], acc_sh.at[dst_b.at[b]], add=True)

    issue(0, 0)

    def body(j, _):
        c = 2 * j
        issue(c + 1, 1)
        finish(c, 0)
        issue(c + 2, 0)
        finish(c + 1, 1)
        return 0

    lax.fori_loop(0, (NCHUNK - 1) // 2, body, 0)
    finish(NCHUNK - 1, 0)

    plsc.subcore_barrier()
    pltpu.sync_copy(acc_sh.at[pl.ds(sid * RPT, RPT)],
                    out_hbm.at[cid, pl.ds(sid * RPT, RPT)])

    @pl.when(sid == 0)
    def _():
        pltpu.sync_copy(acc_sh.at[pl.ds(NS * RPT, TAIL)],
                        out_hbm.at[cid, pl.ds(NS * RPT, TAIL)])


@functools.cache
def _seg_partials():
    return pl.kernel(
        _seg_body,
        out_type=jax.ShapeDtypeStruct((NC, N, D), jnp.float32),
        mesh=plsc.VectorSubcoreMesh(core_axis_name="c", subcore_axis_name="s",
                                    num_cores=NC, num_subcores=NS),
        scratch_types=[
            pltpu.VMEM((EPW,), jnp.int32),
            pltpu.VMEM((2, C), jnp.int32),
            pltpu.VMEM((2, C, D), jnp.float32),
            pltpu.VMEM_SHARED((N, D), jnp.float32),
            pltpu.SemaphoreType.DMA((2,)),
            pltpu.SemaphoreType.DMA((2,)),
        ],
    )


def _bn_elu(y, g, be):
    mean = jnp.mean(y, axis=0, keepdims=True)
    yc = y - mean
    var = jnp.mean(yc * yc, axis=0, keepdims=True)
    yn = yc * lax.rsqrt(var + 1e-5) * g + be
    return jnp.where(yn > 0, yn, jnp.exp(yn) - 1.0)


def _conv_body(x_ref, p_ref, W_ref, b_ref, g_ref, be_ref, o_ref):
    hh = x_ref[...] + p_ref[0] + p_ref[1]
    y = jnp.dot(hh, W_ref[...], preferred_element_type=jnp.float32) + b_ref[...]
    o_ref[...] = _bn_elu(y, g_ref[...], be_ref[...])


def _lin_z(h, W, b, g, be):
    y = jnp.dot(h, W, preferred_element_type=jnp.float32) + b
    return _bn_elu(y, g, be)


def _pool_max(z, mask):
    cols = []
    for t in range(T):
        v = jnp.where(mask, z[:, t:t + 1], -jnp.inf)
        cols.append(jnp.max(v, axis=0).reshape(G, 1))
    return jnp.concatenate(cols, axis=1)


def _heads_body(x_ref, x1_ref, x2_ref, batch_ref, lw_ref,
                lW_ref, lb_ref, lg_ref, lbe_ref,
                out_ref, Z_ref):
    z0 = _lin_z(x_ref[...], lW_ref[0], lb_ref[0], lg_ref[0], lbe_ref[0])
    z1 = _lin_z(x1_ref[...], lW_ref[1], lb_ref[1], lg_ref[1], lbe_ref[1])
    z2 = _lin_z(x2_ref[...], lW_ref[2], lb_ref[2], lg_ref[2], lbe_ref[2])
    z1 = z1 * lw_ref[1:2, 0:1]
    z2 = z2 * lw_ref[2:3, 0:1]

    Z_ref[...] = z0 * lw_ref[0:1, 0:1] + z1 + z2

    gids = lax.broadcasted_iota(jnp.int32, (1, G), 1)
    mask = batch_ref[...] == gids
    out_ref[...] = _pool_max(z0, mask) + _pool_max(z1, mask) + _pool_max(z2, mask)


def kernel(x, edge_index, batch, layer_weights, lin_W, lin_b, lin_g, lin_be,
           conv_W, conv_b, conv_g, conv_be):
    src = edge_index[0]
    dst = edge_index[1]
    zeros = jnp.zeros((N, D), jnp.float32)

    p0 = _seg_partials()(src, dst, zeros, x)

    x1 = pl.pallas_call(
        _conv_body,
        out_shape=jax.ShapeDtypeStruct((N, D), jnp.float32),
    )(x, p0, conv_W[0], conv_b[0].reshape(1, D), conv_g[0].reshape(1, D),
      conv_be[0].reshape(1, D))

    p1 = _seg_partials()(src, dst, zeros, x1)

    x2 = pl.pallas_call(
        _conv_body,
        out_shape=jax.ShapeDtypeStruct((N, D), jnp.float32),
    )(x1, p1, conv_W[1], conv_b[1].reshape(1, D), conv_g[1].reshape(1, D),
      conv_be[1].reshape(1, D))

    out, Z = pl.pallas_call(
        _heads_body,
        out_shape=[
            jax.ShapeDtypeStruct((G, T), jnp.float32),
            jax.ShapeDtypeStruct((N, T), jnp.float32),
        ],
    )(x, x1, x2, batch.reshape(N, 1), layer_weights.reshape(3, 1),
      lin_W, lin_b.reshape(3, 1, T), lin_g.reshape(3, 1, T),
      lin_be.reshape(3, 1, T))

    return out, Z, x2


# speedup vs baseline: 10.3908x; 1.1365x over previous
import functools

import jax
import jax.numpy as jnp
from jax import lax
from jax.experimental import pallas as pl
from jax.experimental.pallas import tpu as pltpu
from jax.experimental.pallas import tpu_sc as plsc

N = 10000
E = 320000
D = 128
T = 10
G = 64

NC = 2
NS = 16
NW = NC * NS
EPW = E // NW
C = 80
NCHUNK = EPW // C
NBUF = 3
RPT = 624
TAIL = N - NS * RPT


def _seg_body(src_hbm, dst_hbm, zeros_hbm, h_hbm, out_hbm,
              src_all, dst_b, rows_b, acc_sh, gsems, isems):
    cid = lax.axis_index("c")
    sid = lax.axis_index("s")
    wid = sid * NC + cid

    pltpu.sync_copy(zeros_hbm.at[pl.ds(sid * RPT, RPT)],
                    acc_sh.at[pl.ds(sid * RPT, RPT)])

    @pl.when(sid == 0)
    def _():
        pltpu.sync_copy(zeros_hbm.at[pl.ds(NS * RPT, TAIL)],
                        acc_sh.at[pl.ds(NS * RPT, TAIL)])

    base = wid * EPW
    pltpu.sync_copy(src_hbm.at[pl.ds(base, EPW)], src_all)
    plsc.subcore_barrier()

    def issue(c, b):
        pltpu.async_copy(dst_hbm.at[pl.ds(base + c * C, C)], dst_b.at[b],
                         isems.at[b])
        pltpu.async_copy(h_hbm.at[src_all.at[pl.ds(c * C, C)]], rows_b.at[b],
                         gsems.at[b])

    def finish(c, b):
        pltpu.make_async_copy(dst_hbm.at[pl.ds(base + c * C, C)], dst_b.at[b],
                              isems.at[b]).wait()
        pltpu.make_async_copy(h_hbm.at[src_all.at[pl.ds(c * C, C)]],
                              rows_b.at[b], gsems.at[b]).wait()
        pltpu.sync_copy(rows_b.at[b], acc_sh.at[dst_b.at[b]], add=True)

    for c in range(NBUF - 1):
        issue(c, c % NBUF)

    NGRP = (NCHUNK - (NBUF - 1)) // NBUF

    def body(j, _):
        c0 = NBUF * j
        for b in range(NBUF):
            issue(c0 + b + NBUF - 1, (b + NBUF - 1) % NBUF)
            finish(c0 + b, b)
        return 0

    lax.fori_loop(0, NGRP, body, 0)
    for c in range(NBUF * NGRP, NCHUNK):
        if c + NBUF - 1 < NCHUNK:
            issue(c + NBUF - 1, (c + NBUF - 1) % NBUF)
        finish(c, c % NBUF)

    plsc.subcore_barrier()
    pltpu.sync_copy(acc_sh.at[pl.ds(sid * RPT, RPT)],
                    out_hbm.at[cid, pl.ds(sid * RPT, RPT)])

    @pl.when(sid == 0)
    def _():
        pltpu.sync_copy(acc_sh.at[pl.ds(NS * RPT, TAIL)],
                        out_hbm.at[cid, pl.ds(NS * RPT, TAIL)])


@functools.cache
def _seg_partials():
    return pl.kernel(
        _seg_body,
        out_type=jax.ShapeDtypeStruct((NC, N, D), jnp.float32),
        mesh=plsc.VectorSubcoreMesh(core_axis_name="c", subcore_axis_name="s",
                                    num_cores=NC, num_subcores=NS),
        scratch_types=[
            pltpu.VMEM((EPW,), jnp.int32),
            pltpu.VMEM((NBUF, C), jnp.int32),
            pltpu.VMEM((NBUF, C, D), jnp.float32),
            pltpu.VMEM_SHARED((N, D), jnp.float32),
            pltpu.SemaphoreType.DMA((NBUF,)),
            pltpu.SemaphoreType.DMA((NBUF,)),
        ],
    )


def _bn_elu(y, g, be):
    mean = jnp.mean(y, axis=0, keepdims=True)
    yc = y - mean
    var = jnp.mean(yc * yc, axis=0, keepdims=True)
    yn = yc * lax.rsqrt(var + 1e-5) * g + be
    return jnp.where(yn > 0, yn, jnp.exp(yn) - 1.0)


def _conv_body(x_ref, p_ref, W_ref, b_ref, g_ref, be_ref, o_ref):
    hh = x_ref[...] + p_ref[0] + p_ref[1]
    y = jnp.dot(hh, W_ref[...], preferred_element_type=jnp.float32) + b_ref[...]
    o_ref[...] = _bn_elu(y, g_ref[...], be_ref[...])


def _lin_z(h, W, b, g, be):
    y = jnp.dot(h, W, preferred_element_type=jnp.float32) + b
    return _bn_elu(y, g, be)


def _pool_max(z, mask):
    cols = []
    for t in range(T):
        v = jnp.where(mask, z[:, t:t + 1], -jnp.inf)
        cols.append(jnp.max(v, axis=0).reshape(G, 1))
    return jnp.concatenate(cols, axis=1)


def _heads_body(x_ref, x1_ref, x2_ref, batch_ref, lw_ref,
                lW_ref, lb_ref, lg_ref, lbe_ref,
                out_ref, Z_ref):
    z0 = _lin_z(x_ref[...], lW_ref[0], lb_ref[0], lg_ref[0], lbe_ref[0])
    z1 = _lin_z(x1_ref[...], lW_ref[1], lb_ref[1], lg_ref[1], lbe_ref[1])
    z2 = _lin_z(x2_ref[...], lW_ref[2], lb_ref[2], lg_ref[2], lbe_ref[2])
    z1 = z1 * lw_ref[1:2, 0:1]
    z2 = z2 * lw_ref[2:3, 0:1]

    Z_ref[...] = z0 * lw_ref[0:1, 0:1] + z1 + z2

    gids = lax.broadcasted_iota(jnp.int32, (1, G), 1)
    mask = batch_ref[...] == gids
    out_ref[...] = _pool_max(z0, mask) + _pool_max(z1, mask) + _pool_max(z2, mask)


def kernel(x, edge_index, batch, layer_weights, lin_W, lin_b, lin_g, lin_be,
           conv_W, conv_b, conv_g, conv_be):
    src = edge_index[0]
    dst = edge_index[1]
    zeros = jnp.zeros((N, D), jnp.float32)

    p0 = _seg_partials()(src, dst, zeros, x)

    x1 = pl.pallas_call(
        _conv_body,
        out_shape=jax.ShapeDtypeStruct((N, D), jnp.float32),
    )(x, p0, conv_W[0], conv_b[0].reshape(1, D), conv_g[0].reshape(1, D),
      conv_be[0].reshape(1, D))

    p1 = _seg_partials()(src, dst, zeros, x1)

    x2 = pl.pallas_call(
        _conv_body,
        out_shape=jax.ShapeDtypeStruct((N, D), jnp.float32),
    )(x1, p1, conv_W[1], conv_b[1].reshape(1, D), conv_g[1].reshape(1, D),
      conv_be[1].reshape(1, D))

    out, Z = pl.pallas_call(
        _heads_body,
        out_shape=[
            jax.ShapeDtypeStruct((G, T), jnp.float32),
            jax.ShapeDtypeStruct((N, T), jnp.float32),
        ],
    )(x, x1, x2, batch.reshape(N, 1), layer_weights.reshape(3, 1),
      lin_W, lin_b.reshape(3, 1, T), lin_g.reshape(3, 1, T),
      lin_be.reshape(3, 1, T))

    return out, Z, x2
